# baseline (speedup 1.0000x reference)
import jax
import jax.numpy as jnp
from jax import lax
from jax.experimental import pallas as pl
from jax.experimental.pallas import tpu as pltpu
from jax.experimental.pallas import tpu_sc as plsc

_B = 4096
_HIST = 50
_NF = 4
_D = 16
_ED = _NF * _D
_NIN = _NF * _ED + _NF * (_NF + 1) // 2
_NPAD = 384
_NW = 32
_BAGS_PW = _B // _NW
_CHUNK_BAGS = 2
_ROWS_PER_CHUNK = _CHUNK_BAGS * _HIST
_ROWS_PAD = 104
_CHUNKS_PW = _BAGS_PW // _CHUNK_BAGS


def _row_vecs(ref, r):
    return [ref[r, pl.ds(16 * k, 16)] for k in range(4)]


_GDN = lax.GatherDimensionNumbers(
    offset_dims=(), collapsed_slice_dims=(0,), start_index_map=(0,))


def _perm16(v, p):
    return lax.gather(v, p[:, None], _GDN, (1,),
                      mode=lax.GatherScatterMode.PROMISE_IN_BOUNDS)


def _lane_perms():
    idx = lax.iota(jnp.int32, 16)
    return [jnp.bitwise_xor(idx, jnp.int32(sh)) for sh in (8, 4, 2, 1)]


def _lane_sum(v, perms):
    for p in perms:
        v = v + _perm16(v, p)
    return v


def _renorm_rows(ref, perms):
    def body(r, carry):
        v = _row_vecs(ref, r)
        ssv = v[0] * v[0] + v[1] * v[1] + v[2] * v[2] + v[3] * v[3]
        sv = jnp.maximum(_lane_sum(ssv, perms), jnp.float32(1e-14))
        iv = lax.bitcast_convert_type(sv, jnp.int32)
        yv = lax.bitcast_convert_type(jnp.int32(0x5F3759DF) - (iv >> 1),
                                      jnp.float32)
        for _ in range(3):
            yv = yv * (jnp.float32(1.5) - jnp.float32(0.5) * sv * yv * yv)
        sc = jnp.minimum(yv, jnp.float32(1.0))
        for k in range(4):
            ref[r, pl.ds(16 * k, 16)] = v[k] * sc
        return carry

    lax.fori_loop(0, _BAGS_PW, body, 0, unroll=2)


def _sc_embed(W_hist, W_aid, W_rid, idxall,
              bag_out, ad_out, rid_out,
              idx_v, aidx_v, ridx_v,
              rows_a, rows_r, hb0, hb1, bagbuf, gsem, sem0, sem1):
    wid = lax.axis_index("s") * 2 + lax.axis_index("c")
    bag0 = wid * _BAGS_PW
    perms = _lane_perms()

    pltpu.sync_copy(idxall.at[pl.ds(wid * _CHUNKS_PW, _CHUNKS_PW)], idx_v)
    pltpu.sync_copy(idxall.at[pl.ds(2048 + wid, 1)], aidx_v)
    pltpu.sync_copy(idxall.at[pl.ds(2048 + _NW + wid, 1)], ridx_v)

    pltpu.async_copy(W_aid.at[aidx_v.at[0]], rows_a, gsem).wait()
    _renorm_rows(rows_a, perms)
    pltpu.async_copy(W_rid.at[ridx_v.at[0]], rows_r, gsem).wait()
    _renorm_rows(rows_r, perms)

    def _start(c, hb, sem):
        pltpu.async_copy(W_hist.at[idx_v.at[c]], hb, sem)

    def _wait(hb, sem):
        pltpu.make_async_copy(W_hist.at[idx_v.at[0]], hb, sem).wait()

    def _compute(c, hb):
        for h in range(_CHUNK_BAGS):
            bl = c * _CHUNK_BAGS + h
            a = _row_vecs(rows_a, bl)

            def row_body(r, acc):
                rr = h * _HIST + r
                hv = _row_vecs(hb, rr)
                ssv = hv[0] * hv[0] + hv[1] * hv[1] + hv[2] * hv[2] + hv[3] * hv[3]
                dtv = hv[0] * a[0] + hv[1] * a[1] + hv[2] * a[2] + hv[3] * a[3]
                ss = _lane_sum(ssv, perms)
                dt = _lane_sum(dtv, perms)
                cc = jnp.minimum(jnp.float32(1.0),
                                 jnp.float32(1.0) / jnp.maximum(ss, jnp.float32(1e-14))) * dt
                return tuple(acc[k] + cc * hv[k] for k in range(4))

            zero = jnp.zeros((16,), jnp.float32)
            acc = lax.fori_loop(0, _HIST, row_body, (zero, zero, zero, zero),
                                unroll=5)
            for k in range(4):
                bagbuf[bl, pl.ds(16 * k, 16)] = acc[k]
                bagbuf[bl, pl.ds(_ED + 16 * k, 16)] = acc[k]

    _start(0, hb0, sem0)

    def pair_body(t, carry):
        _wait(hb0, sem0)
        _start(2 * t + 1, hb1, sem1)
        _compute(2 * t, hb0)
        _wait(hb1, sem1)

        @pl.when(t < _CHUNKS_PW // 2 - 1)
        def _():
            _start(2 * t + 2, hb0, sem0)

        _compute(2 * t + 1, hb1)
        return carry

    lax.fori_loop(0, _CHUNKS_PW // 2, pair_body, 0)

    pltpu.sync_copy(bagbuf, bag_out.at[pl.ds(bag0, _BAGS_PW)])
    pltpu.sync_copy(rows_a, ad_out.at[pl.ds(bag0, _BAGS_PW)])
    pltpu.sync_copy(rows_r, rid_out.at[pl.ds(bag0, _BAGS_PW)])


_sc_call = pl.kernel(
    _sc_embed,
    out_type=[jax.ShapeDtypeStruct((_B, 2 * _ED), jnp.float32)] * 3,
    mesh=plsc.VectorSubcoreMesh(core_axis_name="c", subcore_axis_name="s"),
    scratch_types=[
        pltpu.VMEM((_CHUNKS_PW, 128), jnp.int32),
        pltpu.VMEM((1, _BAGS_PW), jnp.int32),
        pltpu.VMEM((1, _BAGS_PW), jnp.int32),
        pltpu.VMEM((_BAGS_PW, 2 * _ED), jnp.float32),
        pltpu.VMEM((_BAGS_PW, 2 * _ED), jnp.float32),
        pltpu.VMEM((128, 2 * _ED), jnp.float32),
        pltpu.VMEM((128, 2 * _ED), jnp.float32),
        pltpu.VMEM((_BAGS_PW, 2 * _ED), jnp.float32),
        pltpu.SemaphoreType.DMA,
        pltpu.SemaphoreType.DMA,
        pltpu.SemaphoreType.DMA,
    ],
    compiler_params=pltpu.CompilerParams(use_tc_tiling_on_sc=False),
)


def _uid_gather(idx_smem, table_hbm, out_ref, buf, sem):
    def start_body(i, carry):
        r = idx_smem[i]
        pltpu.make_async_copy(table_hbm.at[pl.ds(r, 1), :],
                              buf.at[pl.ds(i, 1), :], sem).start()
        return carry

    lax.fori_loop(0, _B, start_body, 0)

    def wait_body(i, carry):
        pltpu.make_async_copy(table_hbm.at[pl.ds(0, 1), :],
                              buf.at[pl.ds(i, 1), :], sem).wait()
        return carry

    lax.fori_loop(0, _B, wait_body, 0)

    x = buf[...]
    ss = jnp.sum(x * x, axis=1, keepdims=True)
    sc = jnp.minimum(lax.rsqrt(jnp.maximum(ss, jnp.float32(1e-14))),
                     jnp.float32(1.0))
    out_ref[...] = x * sc


_uid_call = pl.pallas_call(
    _uid_gather,
    in_specs=[
        pl.BlockSpec(memory_space=pltpu.SMEM),
        pl.BlockSpec(memory_space=pl.ANY),
    ],
    out_shape=jax.ShapeDtypeStruct((_B, _ED), jnp.float32),
    scratch_shapes=[
        pltpu.VMEM((_B, _ED), jnp.float32),
        pltpu.SemaphoreType.DMA,
    ],
)


def _sigmoid(z):
    return jnp.float32(1.0) / (jnp.float32(1.0) + jnp.exp(-z))


def _dice(y, alpha):
    m = jnp.mean(y, axis=0, keepdims=True)
    v = jnp.mean((y - m) * (y - m), axis=0, keepdims=True)
    p = _sigmoid((y - m) * lax.rsqrt(v + jnp.float32(1e-8)))
    return p * y + (jnp.float32(1.0) - p) * alpha * y


def _tc_head(ue, hb, ae, re_, labels, lw, W1p, b1, W2, b2, W3, b3,
             gamma, beta, al1, al2, loss_ref, s_ref, d_ref):
    parts = [ue[...], hb[...][:, :_ED], ae[...][:, :_ED], re_[...][:, :_ED]]
    terms = []
    for i in range(_NF):
        for j in range(i, _NF):
            t = jnp.sum(parts[i][:, 16 * j:16 * j + 16] *
                        parts[j][:, 16 * i:16 * i + 16], axis=-1, keepdims=True)
            terms.append(t)
    pad = jnp.zeros((_B, _NPAD - _NIN), jnp.float32)
    x = jnp.concatenate(parts + terms + [pad], axis=1)
    m = jnp.mean(x, axis=0, keepdims=True)
    v = jnp.mean((x - m) * (x - m), axis=0, keepdims=True)
    x = (x - m) * lax.rsqrt(v + jnp.float32(1e-5)) * gamma[...] + beta[...]
    h1 = jnp.dot(x, W1p[...], preferred_element_type=jnp.float32) + b1[...]
    d1 = _dice(h1, al1[...])
    h2 = jnp.dot(d1, W2[...], preferred_element_type=jnp.float32) + b2[...]
    d2 = _dice(h2, al2[...])
    s = jnp.dot(d2, W3[...], preferred_element_type=jnp.float32) + b3[...]
    t = labels[...]
    l = jnp.maximum(s, 0.0) - s * t + jnp.log1p(jnp.exp(-jnp.abs(s)))
    loss = jnp.sum(lw[...] * l) / jnp.float32(_B)
    loss_ref[...] = jnp.full((1, 1), loss, jnp.float32)
    s_ref[...] = s
    d_ref[...] = d2


_tc_call = pl.pallas_call(
    _tc_head,
    out_shape=[
        jax.ShapeDtypeStruct((1, 1), jnp.float32),
        jax.ShapeDtypeStruct((_B, 1), jnp.float32),
        jax.ShapeDtypeStruct((_B, _ED), jnp.float32),
    ],
)


def kernel(uid_idx, hist_idx, aid_idx, rid_idx, hist_offsets, labels,
           label_weights, W_uid, W_hist, W_aid, W_rid, W1, b1, W2, b2, W3, b3,
           bn0_gamma, bn0_beta, alpha1, alpha2):
    del hist_offsets
    hidx2d = jnp.pad(
        hist_idx.astype(jnp.int32).reshape(_B * _HIST // _ROWS_PER_CHUNK,
                                           _ROWS_PER_CHUNK),
        ((0, 0), (0, 128 - _ROWS_PER_CHUNK)))
    idxall = jnp.concatenate([
        hidx2d,
        aid_idx.astype(jnp.int32).reshape(_NW, _BAGS_PW),
        rid_idx.astype(jnp.int32).reshape(_NW, _BAGS_PW),
    ], axis=0)
    W_hist2 = jnp.concatenate([W_hist, W_hist], axis=1)
    W_aid2 = jnp.concatenate([W_aid, W_aid], axis=1)
    W_rid2 = jnp.concatenate([W_rid, W_rid], axis=1)
    hbag, ae, re_ = _sc_call(W_hist2, W_aid2, W_rid2, idxall)
    ue = _uid_call(uid_idx.astype(jnp.int32), W_uid)
    W1p = jnp.pad(W1, ((0, _NPAD - _NIN), (0, 0)))
    gamma = jnp.pad(bn0_gamma, (0, _NPAD - _NIN)).reshape(1, -1)
    beta = jnp.pad(bn0_beta, (0, _NPAD - _NIN)).reshape(1, -1)
    loss, s, d = _tc_call(ue, hbag, ae, re_, labels.reshape(-1, 1),
                          label_weights.reshape(-1, 1), W1p,
                          b1.reshape(1, -1), W2, b2.reshape(1, -1), W3,
                          b3.reshape(1, -1), gamma, beta,
                          alpha1.reshape(1, -1), alpha2.reshape(1, -1))
    ml = loss.reshape(())
    return (ml, s.reshape(-1), ml, d)


# speedup vs baseline: 3.1003x; 3.1003x over previous
import jax
import jax.numpy as jnp
from jax import lax
from jax.experimental import pallas as pl
from jax.experimental.pallas import tpu as pltpu
from jax.experimental.pallas import tpu_sc as plsc

_B = 4096
_HIST = 50
_NF = 4
_D = 16
_ED = _NF * _D
_NIN = _NF * _ED + _NF * (_NF + 1) // 2
_NPAD = 384
_NW = 32
_BAGS_PW = _B // _NW
_CHUNK_BAGS = 2
_ROWS_PER_CHUNK = _CHUNK_BAGS * _HIST
_ROWS_PAD = 104
_CHUNKS_PW = _BAGS_PW // _CHUNK_BAGS


def _row_vecs(ref, r):
    return [ref[r, pl.ds(16 * k, 16)] for k in range(4)]


_GDN = lax.GatherDimensionNumbers(
    offset_dims=(), collapsed_slice_dims=(0,), start_index_map=(0,))


def _perm16(v, p):
    return lax.gather(v, p[:, None], _GDN, (1,),
                      mode=lax.GatherScatterMode.PROMISE_IN_BOUNDS)


def _lane_perms():
    idx = lax.iota(jnp.int32, 16)
    return [jnp.bitwise_xor(idx, jnp.int32(sh)) for sh in (8, 4, 2, 1)]


def _lane_sum(v, perms):
    for p in perms:
        v = v + _perm16(v, p)
    return v


def _renorm_rows(ref, perms):
    def body(r, carry):
        v = _row_vecs(ref, r)
        ssv = v[0] * v[0] + v[1] * v[1] + v[2] * v[2] + v[3] * v[3]
        sv = jnp.maximum(_lane_sum(ssv, perms), jnp.float32(1e-14))
        iv = lax.bitcast_convert_type(sv, jnp.int32)
        yv = lax.bitcast_convert_type(jnp.int32(0x5F3759DF) - (iv >> 1),
                                      jnp.float32)
        for _ in range(3):
            yv = yv * (jnp.float32(1.5) - jnp.float32(0.5) * sv * yv * yv)
        sc = jnp.minimum(yv, jnp.float32(1.0))
        for k in range(4):
            ref[r, pl.ds(16 * k, 16)] = v[k] * sc
        return carry

    lax.fori_loop(0, _BAGS_PW, body, 0, unroll=2)


def _sc_embed(W_hist, W_aid, W_rid, idxall,
              bag_out, ad_out, rid_out,
              idx_v, aidx_v, ridx_v,
              rows_a, rows_r, hb0, hb1, bagbuf, gsem, sem0, sem1):
    wid = lax.axis_index("s") * 2 + lax.axis_index("c")
    bag0 = wid * _BAGS_PW
    perms = _lane_perms()

    pltpu.sync_copy(idxall.at[pl.ds(wid * _CHUNKS_PW, _CHUNKS_PW)], idx_v)
    pltpu.sync_copy(idxall.at[pl.ds(2048 + wid, 1)], aidx_v)
    pltpu.sync_copy(idxall.at[pl.ds(2048 + _NW + wid, 1)], ridx_v)

    pltpu.async_copy(W_aid.at[aidx_v.at[0]], rows_a, gsem).wait()
    _renorm_rows(rows_a, perms)
    pltpu.async_copy(W_rid.at[ridx_v.at[0]], rows_r, gsem).wait()
    _renorm_rows(rows_r, perms)

    def _start(c, hb, sem):
        pltpu.async_copy(W_hist.at[idx_v.at[c, pl.ds(0, _ROWS_PAD)]], hb, sem)

    def _wait(hb, sem):
        pltpu.make_async_copy(W_hist.at[idx_v.at[0, pl.ds(0, _ROWS_PAD)]], hb,
                              sem).wait()

    def _start4(t, hb, sem):
        for j in range(4):
            pltpu.async_copy(
                W_hist.at[idx_v.at[4 * t + j, pl.ds(0, _ROWS_PAD)]],
                hb.at[pl.ds(_ROWS_PAD * j, _ROWS_PAD)], sem)

    def _wait4(hb, sem):
        pltpu.make_async_copy(W_hist.at[pl.ds(0, 4 * _ROWS_PAD)], hb,
                              sem).wait()

    def _compute2(c, hb, roff):
        for h in range(_CHUNK_BAGS):
            bl = c * _CHUNK_BAGS + h
            a = _row_vecs(rows_a, bl)

            def row_body(r, acc):
                rr = roff + h * _HIST + r
                hv = _row_vecs(hb, rr)
                ssv = hv[0] * hv[0] + hv[1] * hv[1] + hv[2] * hv[2] + hv[3] * hv[3]
                dtv = hv[0] * a[0] + hv[1] * a[1] + hv[2] * a[2] + hv[3] * a[3]
                ss = _lane_sum(ssv, perms)
                dt = _lane_sum(dtv, perms)
                cc = jnp.minimum(jnp.float32(1.0),
                                 jnp.float32(1.0) / jnp.maximum(ss, jnp.float32(1e-14))) * dt
                return tuple(acc[k] + cc * hv[k] for k in range(4))

            zero = jnp.zeros((16,), jnp.float32)
            acc = lax.fori_loop(0, _HIST, row_body, (zero, zero, zero, zero),
                                unroll=5)
            for k in range(4):
                bagbuf[bl, pl.ds(16 * k, 16)] = acc[k]
                bagbuf[bl, pl.ds(_ED + 16 * k, 16)] = acc[k]

    def super_body(t, carry):
        _start4(t, hb0, sem0)
        _wait4(hb0, sem0)
        for j in range(4):
            _compute2(4 * t + j, hb0, _ROWS_PAD * j)
        return carry

    lax.fori_loop(0, _CHUNKS_PW // 4, super_body, 0)

    pltpu.sync_copy(bagbuf, bag_out.at[pl.ds(bag0, _BAGS_PW)])
    pltpu.sync_copy(rows_a, ad_out.at[pl.ds(bag0, _BAGS_PW)])
    pltpu.sync_copy(rows_r, rid_out.at[pl.ds(bag0, _BAGS_PW)])


_sc_call = pl.kernel(
    _sc_embed,
    out_type=[jax.ShapeDtypeStruct((_B, 2 * _ED), jnp.float32)] * 3,
    mesh=plsc.VectorSubcoreMesh(core_axis_name="c", subcore_axis_name="s"),
    scratch_types=[
        pltpu.VMEM((_CHUNKS_PW, 128), jnp.int32),
        pltpu.VMEM((1, _BAGS_PW), jnp.int32),
        pltpu.VMEM((1, _BAGS_PW), jnp.int32),
        pltpu.VMEM((_BAGS_PW, 2 * _ED), jnp.float32),
        pltpu.VMEM((_BAGS_PW, 2 * _ED), jnp.float32),
        pltpu.VMEM((4 * _ROWS_PAD, 2 * _ED), jnp.float32),
        pltpu.VMEM((8, 2 * _ED), jnp.float32),
        pltpu.VMEM((_BAGS_PW, 2 * _ED), jnp.float32),
        pltpu.SemaphoreType.DMA,
        pltpu.SemaphoreType.DMA,
        pltpu.SemaphoreType.DMA,
    ],
    compiler_params=pltpu.CompilerParams(use_tc_tiling_on_sc=False),
)


def _uid_gather(idx_smem, table_hbm, out_ref, buf, sem):
    def start_body(i, carry):
        r = idx_smem[i]
        pltpu.make_async_copy(table_hbm.at[pl.ds(r, 1), :],
                              buf.at[pl.ds(i, 1), :], sem).start()
        return carry

    lax.fori_loop(0, _B, start_body, 0)

    def wait_body(i, carry):
        pltpu.make_async_copy(table_hbm.at[pl.ds(0, 1), :],
                              buf.at[pl.ds(i, 1), :], sem).wait()
        return carry

    lax.fori_loop(0, _B, wait_body, 0)

    x = buf[...]
    ss = jnp.sum(x * x, axis=1, keepdims=True)
    sc = jnp.minimum(lax.rsqrt(jnp.maximum(ss, jnp.float32(1e-14))),
                     jnp.float32(1.0))
    out_ref[...] = x * sc


_uid_call = pl.pallas_call(
    _uid_gather,
    in_specs=[
        pl.BlockSpec(memory_space=pltpu.SMEM),
        pl.BlockSpec(memory_space=pl.ANY),
    ],
    out_shape=jax.ShapeDtypeStruct((_B, _ED), jnp.float32),
    scratch_shapes=[
        pltpu.VMEM((_B, _ED), jnp.float32),
        pltpu.SemaphoreType.DMA,
    ],
)


def _sigmoid(z):
    return jnp.float32(1.0) / (jnp.float32(1.0) + jnp.exp(-z))


def _dice(y, alpha):
    m = jnp.mean(y, axis=0, keepdims=True)
    v = jnp.mean((y - m) * (y - m), axis=0, keepdims=True)
    p = _sigmoid((y - m) * lax.rsqrt(v + jnp.float32(1e-8)))
    return p * y + (jnp.float32(1.0) - p) * alpha * y


def _tc_head(ue, hb, ae, re_, labels, lw, W1p, b1, W2, b2, W3, b3,
             gamma, beta, al1, al2, loss_ref, s_ref, d_ref):
    parts = [ue[...], hb[...][:, :_ED], ae[...][:, :_ED], re_[...][:, :_ED]]
    terms = []
    for i in range(_NF):
        for j in range(i, _NF):
            t = jnp.sum(parts[i][:, 16 * j:16 * j + 16] *
                        parts[j][:, 16 * i:16 * i + 16], axis=-1, keepdims=True)
            terms.append(t)
    pad = jnp.zeros((_B, _NPAD - _NIN), jnp.float32)
    x = jnp.concatenate(parts + terms + [pad], axis=1)
    m = jnp.mean(x, axis=0, keepdims=True)
    v = jnp.mean((x - m) * (x - m), axis=0, keepdims=True)
    x = (x - m) * lax.rsqrt(v + jnp.float32(1e-5)) * gamma[...] + beta[...]
    h1 = jnp.dot(x, W1p[...], preferred_element_type=jnp.float32) + b1[...]
    d1 = _dice(h1, al1[...])
    h2 = jnp.dot(d1, W2[...], preferred_element_type=jnp.float32) + b2[...]
    d2 = _dice(h2, al2[...])
    s = jnp.dot(d2, W3[...], preferred_element_type=jnp.float32) + b3[...]
    t = labels[...]
    l = jnp.maximum(s, 0.0) - s * t + jnp.log1p(jnp.exp(-jnp.abs(s)))
    loss = jnp.sum(lw[...] * l) / jnp.float32(_B)
    loss_ref[...] = jnp.full((1, 1), loss, jnp.float32)
    s_ref[...] = s
    d_ref[...] = d2


_tc_call = pl.pallas_call(
    _tc_head,
    out_shape=[
        jax.ShapeDtypeStruct((1, 1), jnp.float32),
        jax.ShapeDtypeStruct((_B, 1), jnp.float32),
        jax.ShapeDtypeStruct((_B, _ED), jnp.float32),
    ],
)


def kernel(uid_idx, hist_idx, aid_idx, rid_idx, hist_offsets, labels,
           label_weights, W_uid, W_hist, W_aid, W_rid, W1, b1, W2, b2, W3, b3,
           bn0_gamma, bn0_beta, alpha1, alpha2):
    del hist_offsets
    hidx2d = jnp.pad(
        hist_idx.astype(jnp.int32).reshape(_B * _HIST // _ROWS_PER_CHUNK,
                                           _ROWS_PER_CHUNK),
        ((0, 0), (0, 128 - _ROWS_PER_CHUNK)))
    idxall = jnp.concatenate([
        hidx2d,
        aid_idx.astype(jnp.int32).reshape(_NW, _BAGS_PW),
        rid_idx.astype(jnp.int32).reshape(_NW, _BAGS_PW),
    ], axis=0)
    W_hist2 = jnp.concatenate([W_hist, W_hist], axis=1)
    W_aid2 = jnp.concatenate([W_aid, W_aid], axis=1)
    W_rid2 = jnp.concatenate([W_rid, W_rid], axis=1)
    hbag, ae, re_ = _sc_call(W_hist2, W_aid2, W_rid2, idxall)
    ue = _uid_call(uid_idx.astype(jnp.int32), W_uid)
    W1p = jnp.pad(W1, ((0, _NPAD - _NIN), (0, 0)))
    gamma = jnp.pad(bn0_gamma, (0, _NPAD - _NIN)).reshape(1, -1)
    beta = jnp.pad(bn0_beta, (0, _NPAD - _NIN)).reshape(1, -1)
    loss, s, d = _tc_call(ue, hbag, ae, re_, labels.reshape(-1, 1),
                          label_weights.reshape(-1, 1), W1p,
                          b1.reshape(1, -1), W2, b2.reshape(1, -1), W3,
                          b3.reshape(1, -1), gamma, beta,
                          alpha1.reshape(1, -1), alpha2.reshape(1, -1))
    ml = loss.reshape(())
    return (ml, s.reshape(-1), ml, d)


# speedup vs baseline: 3.5880x; 1.1573x over previous
import jax
import jax.numpy as jnp
from jax import lax
from jax.experimental import pallas as pl
from jax.experimental.pallas import tpu as pltpu
from jax.experimental.pallas import tpu_sc as plsc

_B = 4096
_HIST = 50
_NF = 4
_D = 16
_ED = _NF * _D
_NIN = _NF * _ED + _NF * (_NF + 1) // 2
_NPAD = 384
_VRID = 1000
_VRPAD = 1024
_NW = 32
_BAGS_PW = _B // _NW
_CHUNK_BAGS = 2
_ROWS_PER_CHUNK = _CHUNK_BAGS * _HIST
_ROWS_PAD = 104
_CHUNKS_PW = _BAGS_PW // _CHUNK_BAGS


def _row_vecs(ref, r):
    return [ref[r, pl.ds(16 * k, 16)] for k in range(4)]


_GDN = lax.GatherDimensionNumbers(
    offset_dims=(), collapsed_slice_dims=(0,), start_index_map=(0,))


def _perm16(v, p):
    return lax.gather(v, p[:, None], _GDN, (1,),
                      mode=lax.GatherScatterMode.PROMISE_IN_BOUNDS)


def _lane_perms():
    idx = lax.iota(jnp.int32, 16)
    return [jnp.bitwise_xor(idx, jnp.int32(sh)) for sh in (8, 4, 2, 1)]


def _lane_sum(v, perms):
    for p in perms:
        v = v + _perm16(v, p)
    return v


def _renorm_rows(ref, perms):
    def body(r, carry):
        v = _row_vecs(ref, r)
        ssv = v[0] * v[0] + v[1] * v[1] + v[2] * v[2] + v[3] * v[3]
        sv = jnp.maximum(_lane_sum(ssv, perms), jnp.float32(1e-14))
        iv = lax.bitcast_convert_type(sv, jnp.int32)
        yv = lax.bitcast_convert_type(jnp.int32(0x5F3759DF) - (iv >> 1),
                                      jnp.float32)
        for _ in range(3):
            yv = yv * (jnp.float32(1.5) - jnp.float32(0.5) * sv * yv * yv)
        sc = jnp.minimum(yv, jnp.float32(1.0))
        for k in range(4):
            ref[r, pl.ds(16 * k, 16)] = v[k] * sc
        return carry

    lax.fori_loop(0, _BAGS_PW, body, 0, unroll=2)


def _sc_embed(W_hist, W_aid, idxall,
              bag_out, ad_out,
              idx_v, aidx_v, rows_a, hb0, hb1, bagbuf, gsem, sem0, sem1):
    wid = lax.axis_index("s") * 2 + lax.axis_index("c")
    bag0 = wid * _BAGS_PW
    perms = _lane_perms()

    pltpu.sync_copy(idxall.at[pl.ds(wid * _CHUNKS_PW, _CHUNKS_PW)], idx_v)
    pltpu.sync_copy(idxall.at[pl.ds(2048 + wid, 1)], aidx_v)

    pltpu.async_copy(W_aid.at[aidx_v.at[0, pl.ds(0, 64)]],
                     rows_a.at[pl.ds(0, 64)], gsem)
    pltpu.async_copy(W_aid.at[aidx_v.at[0, pl.ds(64, 64)]],
                     rows_a.at[pl.ds(64, 64)], gsem)
    pltpu.make_async_copy(W_aid.at[pl.ds(0, _BAGS_PW)], rows_a, gsem).wait()
    _renorm_rows(rows_a, perms)

    def _start(c, hb, sem):
        pltpu.async_copy(W_hist.at[idx_v.at[c, pl.ds(0, _ROWS_PAD)]], hb, sem)

    def _wait(hb, sem):
        pltpu.make_async_copy(W_hist.at[pl.ds(0, _ROWS_PAD)], hb, sem).wait()

    def _compute(c, hb):
        for h in range(_CHUNK_BAGS):
            bl = c * _CHUNK_BAGS + h
            a = _row_vecs(rows_a, bl)

            def row_body(r, acc):
                rr = h * _HIST + r
                hv = _row_vecs(hb, rr)
                ssv = hv[0] * hv[0] + hv[1] * hv[1] + hv[2] * hv[2] + hv[3] * hv[3]
                dtv = hv[0] * a[0] + hv[1] * a[1] + hv[2] * a[2] + hv[3] * a[3]
                ss = _lane_sum(ssv, perms)
                dt = _lane_sum(dtv, perms)
                cc = jnp.minimum(jnp.float32(1.0),
                                 jnp.float32(1.0) / jnp.maximum(ss, jnp.float32(1e-14))) * dt
                return tuple(acc[k] + cc * hv[k] for k in range(4))

            zero = jnp.zeros((16,), jnp.float32)
            acc = lax.fori_loop(0, _HIST, row_body, (zero, zero, zero, zero),
                                unroll=5)
            for k in range(4):
                bagbuf[bl, pl.ds(16 * k, 16)] = acc[k]

    _start(0, hb0, sem0)

    def pair_body(t, carry):
        _wait(hb0, sem0)
        _start(2 * t + 1, hb1, sem1)
        _compute(2 * t, hb0)
        _wait(hb1, sem1)

        @pl.when(t < _CHUNKS_PW // 2 - 1)
        def _():
            _start(2 * t + 2, hb0, sem0)

        _compute(2 * t + 1, hb1)
        return carry

    lax.fori_loop(0, _CHUNKS_PW // 2, pair_body, 0)

    pltpu.sync_copy(bagbuf, bag_out.at[pl.ds(bag0, _BAGS_PW)])
    pltpu.sync_copy(rows_a, ad_out.at[pl.ds(bag0, _BAGS_PW)])


_sc_call = pl.kernel(
    _sc_embed,
    out_type=[
        jax.ShapeDtypeStruct((_B, _ED), jnp.float32),
        jax.ShapeDtypeStruct((_B, 2 * _ED), jnp.float32),
    ],
    mesh=plsc.VectorSubcoreMesh(core_axis_name="c", subcore_axis_name="s"),
    scratch_types=[
        pltpu.VMEM((_CHUNKS_PW, 128), jnp.int32),
        pltpu.VMEM((1, _BAGS_PW), jnp.int32),
        pltpu.VMEM((_BAGS_PW, 2 * _ED), jnp.float32),
        pltpu.VMEM((_ROWS_PAD, _ED), jnp.float32),
        pltpu.VMEM((_ROWS_PAD, _ED), jnp.float32),
        pltpu.VMEM((_BAGS_PW, _ED), jnp.float32),
        pltpu.SemaphoreType.DMA,
        pltpu.SemaphoreType.DMA,
        pltpu.SemaphoreType.DMA,
    ],
    compiler_params=pltpu.CompilerParams(use_tc_tiling_on_sc=False),
)


def _uid_gather(idx_smem, table_hbm, out_ref, buf, sem):
    def start_body(i, carry):
        r = idx_smem[i]
        pltpu.make_async_copy(table_hbm.at[pl.ds(r, 1), :],
                              buf.at[pl.ds(i, 1), :], sem).start()
        return carry

    lax.fori_loop(0, _B, start_body, 0)

    def wait_body(i, carry):
        pltpu.make_async_copy(table_hbm.at[pl.ds(0, 1), :],
                              buf.at[pl.ds(i, 1), :], sem).wait()
        return carry

    lax.fori_loop(0, _B, wait_body, 0)

    x = buf[...]
    ss = jnp.sum(x * x, axis=1, keepdims=True)
    sc = jnp.minimum(lax.rsqrt(jnp.maximum(ss, jnp.float32(1e-14))),
                     jnp.float32(1.0))
    out_ref[...] = x * sc


_uid_call = pl.pallas_call(
    _uid_gather,
    in_specs=[
        pl.BlockSpec(memory_space=pltpu.SMEM),
        pl.BlockSpec(memory_space=pl.ANY),
    ],
    out_shape=jax.ShapeDtypeStruct((_B, _ED), jnp.float32),
    scratch_shapes=[
        pltpu.VMEM((_B, _ED), jnp.float32),
        pltpu.SemaphoreType.DMA,
    ],
)


def _sigmoid(z):
    return jnp.float32(1.0) / (jnp.float32(1.0) + jnp.exp(-z))


def _dice(y, alpha):
    m = jnp.mean(y, axis=0, keepdims=True)
    v = jnp.mean((y - m) * (y - m), axis=0, keepdims=True)
    p = _sigmoid((y - m) * lax.rsqrt(v + jnp.float32(1e-8)))
    return p * y + (jnp.float32(1.0) - p) * alpha * y


def _tc_head(ue, hb, ae, ridx, W_ridp, labels, lw, W1p, b1, W2, b2, W3, b3,
             gamma, beta, al1, al2, loss_ref, s_ref, d_ref):
    iot = lax.broadcasted_iota(jnp.int32, (_B, _VRPAD), 1)
    oh = (ridx[...] == iot).astype(jnp.float32)
    rr = jnp.dot(oh, W_ridp[...], preferred_element_type=jnp.float32)
    ssr = jnp.sum(rr * rr, axis=1, keepdims=True)
    scr = jnp.minimum(lax.rsqrt(jnp.maximum(ssr, jnp.float32(1e-14))),
                      jnp.float32(1.0))
    re_ = rr * scr

    parts = [ue[...], hb[...], ae[...][:, :_ED], re_]
    terms = []
    for i in range(_NF):
        for j in range(i, _NF):
            t = jnp.sum(parts[i][:, 16 * j:16 * j + 16] *
                        parts[j][:, 16 * i:16 * i + 16], axis=-1, keepdims=True)
            terms.append(t)
    pad = jnp.zeros((_B, _NPAD - _NIN), jnp.float32)
    x = jnp.concatenate(parts + terms + [pad], axis=1)
    m = jnp.mean(x, axis=0, keepdims=True)
    v = jnp.mean((x - m) * (x - m), axis=0, keepdims=True)
    x = (x - m) * lax.rsqrt(v + jnp.float32(1e-5)) * gamma[...] + beta[...]
    h1 = jnp.dot(x, W1p[...], preferred_element_type=jnp.float32) + b1[...]
    d1 = _dice(h1, al1[...])
    h2 = jnp.dot(d1, W2[...], preferred_element_type=jnp.float32) + b2[...]
    d2 = _dice(h2, al2[...])
    s = jnp.dot(d2, W3[...], preferred_element_type=jnp.float32) + b3[...]
    t = labels[...]
    l = jnp.maximum(s, 0.0) - s * t + jnp.log1p(jnp.exp(-jnp.abs(s)))
    loss = jnp.sum(lw[...] * l) / jnp.float32(_B)
    loss_ref[...] = jnp.full((1, 1), loss, jnp.float32)
    s_ref[...] = s
    d_ref[...] = d2


_tc_call = pl.pallas_call(
    _tc_head,
    out_shape=[
        jax.ShapeDtypeStruct((1, 1), jnp.float32),
        jax.ShapeDtypeStruct((_B, 1), jnp.float32),
        jax.ShapeDtypeStruct((_B, _ED), jnp.float32),
    ],
)


def kernel(uid_idx, hist_idx, aid_idx, rid_idx, hist_offsets, labels,
           label_weights, W_uid, W_hist, W_aid, W_rid, W1, b1, W2, b2, W3, b3,
           bn0_gamma, bn0_beta, alpha1, alpha2):
    del hist_offsets
    hidx2d = jnp.pad(
        hist_idx.astype(jnp.int32).reshape(_B * _HIST // _ROWS_PER_CHUNK,
                                           _ROWS_PER_CHUNK),
        ((0, 0), (0, 128 - _ROWS_PER_CHUNK)))
    idxall = jnp.concatenate([
        hidx2d,
        aid_idx.astype(jnp.int32).reshape(_NW, _BAGS_PW),
    ], axis=0)
    W_aid2 = jnp.concatenate([W_aid, W_aid], axis=1)
    hbag, ae = _sc_call(W_hist, W_aid2, idxall)
    ue = _uid_call(uid_idx.astype(jnp.int32), W_uid)
    W_ridp = jnp.pad(W_rid, ((0, _VRPAD - _VRID), (0, 0)))
    W1p = jnp.pad(W1, ((0, _NPAD - _NIN), (0, 0)))
    gamma = jnp.pad(bn0_gamma, (0, _NPAD - _NIN)).reshape(1, -1)
    beta = jnp.pad(bn0_beta, (0, _NPAD - _NIN)).reshape(1, -1)
    loss, s, d = _tc_call(ue, hbag, ae, rid_idx.astype(jnp.int32).reshape(-1, 1),
                          W_ridp, labels.reshape(-1, 1),
                          label_weights.reshape(-1, 1), W1p,
                          b1.reshape(1, -1), W2, b2.reshape(1, -1), W3,
                          b3.reshape(1, -1), gamma, beta,
                          alpha1.reshape(1, -1), alpha2.reshape(1, -1))
    ml = loss.reshape(())
    return (ml, s.reshape(-1), ml, d)
